# Initial kernel scaffold; baseline (speedup 1.0000x reference)
import jax
import jax.numpy as jnp
from jax.experimental import pallas as pl


def kernel(support_indices, query_indices, item_idx, user_table, item_table):
    raise NotImplementedError("write your pallas kernel here")


# speedup vs baseline: 1.9942x; 1.9942x over previous
import functools

import jax
import jax.numpy as jnp
from jax import lax
from jax.experimental import pallas as pl
from jax.experimental.pallas import tpu as pltpu
from jax.experimental.pallas import tpu_sc as plsc

B = 1024
D = 128
N_CAND = 16384

NC = 2
NS = 16
NW = NC * NS

USERS_PER_W = B // NW
ITEMS_PER_W = N_CAND // NW
IDX_CHUNK = 128


def _gather_body(qidx_hbm, iidx_hbm, user_hbm, item_hbm, u_out, it_out,
                 qi_v, u_v, ii_v, it_v, sem):
    wid = lax.axis_index("s") * NC + lax.axis_index("c")

    ub = wid * USERS_PER_W
    pltpu.sync_copy(qidx_hbm.at[pl.ds(ub, USERS_PER_W)], qi_v)
    pltpu.async_copy(user_hbm.at[qi_v], u_v, sem).wait()
    pltpu.sync_copy(u_v, u_out.at[pl.ds(ub, USERS_PER_W)])

    ib = wid * ITEMS_PER_W
    pltpu.sync_copy(iidx_hbm.at[pl.ds(ib, ITEMS_PER_W)], ii_v)
    copies = []
    for j in range(ITEMS_PER_W // IDX_CHUNK):
        copies.append(pltpu.async_copy(
            item_hbm.at[ii_v.at[pl.ds(j * IDX_CHUNK, IDX_CHUNK)]],
            it_v.at[pl.ds(j * IDX_CHUNK, IDX_CHUNK)],
            sem,
        ))
    for c in copies:
        c.wait()
    pltpu.sync_copy(it_v, it_out.at[pl.ds(ib, ITEMS_PER_W)])


_gather = functools.partial(
    pl.kernel,
    mesh=plsc.VectorSubcoreMesh(core_axis_name="c", subcore_axis_name="s"),
    out_type=[
        jax.ShapeDtypeStruct((B, D), jnp.float32),
        jax.ShapeDtypeStruct((N_CAND, D), jnp.float32),
    ],
    scratch_types=[
        pltpu.VMEM((USERS_PER_W,), jnp.int32),
        pltpu.VMEM((USERS_PER_W, D), jnp.float32),
        pltpu.VMEM((ITEMS_PER_W,), jnp.int32),
        pltpu.VMEM((ITEMS_PER_W, D), jnp.float32),
        pltpu.SemaphoreType.DMA,
    ],
)(_gather_body)


TILE_N = 2048


def _mm_body(u_ref, it_ref, o_ref):
    o_ref[...] = lax.dot_general(
        u_ref[...], it_ref[...],
        dimension_numbers=(((1,), (1,)), ((), ())),
        preferred_element_type=jnp.float32,
    )


def kernel(support_indices, query_indices, item_idx, user_table, item_table):
    del support_indices
    qidx = query_indices.astype(jnp.int32)
    iidx = item_idx.astype(jnp.int32)

    u, it = _gather(qidx, iidx, user_table, item_table)

    scores = pl.pallas_call(
        _mm_body,
        grid=(N_CAND // TILE_N,),
        in_specs=[
            pl.BlockSpec((B, D), lambda j: (0, 0)),
            pl.BlockSpec((TILE_N, D), lambda j: (j, 0)),
        ],
        out_specs=pl.BlockSpec((B, TILE_N), lambda j: (0, j)),
        out_shape=jax.ShapeDtypeStruct((B, N_CAND), jnp.float32),
    )(u, it)
    return scores
